# baseline (speedup 1.0000x reference)
import jax
import jax.numpy as jnp
from jax import lax
from jax.experimental import pallas as pl
from jax.experimental.pallas import tpu as pltpu
from jax.experimental.pallas import tpu_sc as plsc

B = 1024
L = 1024
N = 16
NC, NS, LANES = 2, 16, 16
NW = NC * NS
SPLIT = 512
SC_PROBE = False
RPW = max(SPLIT, NW) // NW
NG = RPW // LANES
TBR = 256
TNB = (B - SPLIT) // TBR


def _tree_sum(xs):
    while len(xs) > 1:
        xs = [xs[i] + xs[i + 1] for i in range(0, len(xs), 2)]
    return xs[0]


def _sc_body(idx_hbm, eps_hbm, bvec_hbm, out_hbm, idx_v, eps_v, out_v,
             eps_sh, bvec_v):
    sid = lax.axis_index("s")
    wid = sid * NC + lax.axis_index("c")
    pltpu.sync_copy(idx_hbm.at[pl.ds(wid * (RPW * L), RPW * L)], idx_v)
    pltpu.sync_copy(bvec_hbm, bvec_v)

    @pl.when(sid == 0)
    def _():
        pltpu.sync_copy(eps_hbm, eps_sh)
    plsc.subcore_barrier()
    pltpu.sync_copy(eps_sh, eps_v)

    def _epilogue(chosen, other, acc):
        m = jnp.maximum(chosen, other)
        mn = jnp.minimum(chosen, other)
        y = jnp.exp(2.0 * (mn - m))
        z = y / (2.0 + y)
        z2 = z * z
        log1p = 2.0 * z * (1.0 + z2 * (1.0 / 3 + z2 * (1.0 / 5 + z2 * (
            1.0 / 7))))
        return acc + (chosen - (m + 0.5 * log1p))

    def _esel_base(t, bvec):
        srow = plsc.load_gather(idx_v, [bvec + t])
        return srow * N + t * (2 * N)

    def _es_row(t):
        return (eps_v[pl.ds(t * (2 * N), N)] +
                eps_v[pl.ds(t * (2 * N) + N, N)])

    for g in range(NG):
        bvec = bvec_v[pl.ds(g * LANES, LANES)]

        def step(t, carry, g=g, bvec=bvec):
            acc = carry[0]
            base = carry[1]
            ES = carry[2]
            P = list(carry[3:])
            tsum, csum = [], []
            for n in range(N):
                esel = plsc.load_gather(eps_v, [base + n])
                tsum.append(P[n] * ES[n])
                P[n] = P[n] * esel
                csum.append(P[n])
            base_n = _esel_base(t + 1, bvec)
            ES_n = _es_row(t + 1)
            chosen = _tree_sum(csum)
            other = _tree_sum(tsum) - chosen
            acc = _epilogue(chosen, other, acc)
            return (acc, base_n, ES_n, *P)

        ones = jnp.ones((LANES,), jnp.float32)
        zeros = jnp.zeros((LANES,), jnp.float32)
        carry = lax.fori_loop(0, L - 1, step,
                              (zeros, _esel_base(0, bvec), _es_row(0))
                              + (ones,) * N,
                              unroll=8)
        acc, base, ES = carry[0], carry[1], carry[2]
        P = list(carry[3:])
        tsum, csum = [], []
        for n in range(N):
            esel = plsc.load_gather(eps_v, [base + n])
            tsum.append(P[n] * ES[n])
            csum.append(P[n] * esel)
        chosen = _tree_sum(csum)
        other = _tree_sum(tsum) - chosen
        acc = _epilogue(chosen, other, acc)
        out_v[pl.ds(g * LANES, LANES)] = acc

    pltpu.sync_copy(out_v, out_hbm.at[pl.ds(wid * RPW, RPW)])


def _sc_call(idx_r, eps_r, bvec):
    f = pl.kernel(
        _sc_body,
        out_type=jax.ShapeDtypeStruct((SPLIT,), jnp.float32),
        mesh=plsc.VectorSubcoreMesh(core_axis_name="c", subcore_axis_name="s"),
        compiler_params=pltpu.CompilerParams(needs_layout_passes=False, skip_device_barrier=True),
        scratch_types=[
            pltpu.VMEM((L * RPW,), jnp.int32),
            pltpu.VMEM((L * 2 * N,), jnp.float32),
            pltpu.VMEM((RPW,), jnp.float32),
            pltpu.VMEM_SHARED((L * 2 * N,), jnp.float32),
            pltpu.VMEM((NG * LANES,), jnp.int32),
        ],
    )
    return f(idx_r, eps_r, bvec)


def _tc_body(idx_ref, e0_ref, e1_ref, out_ref):
    is1 = idx_ref[...] > 0
    ls0 = jnp.zeros((TBR, L), jnp.float32)
    ls1 = jnp.zeros((TBR, L), jnp.float32)
    for n in range(N):
        e0 = e0_ref[n, :].reshape(1, L)
        e1 = e1_ref[n, :].reshape(1, L)
        x = jnp.where(is1, e1, e0)
        x = jnp.concatenate(
            [jnp.ones((TBR, 1), jnp.float32), x[:, :L - 1]], axis=1)
        d = 1
        while d < L:
            xs = jnp.concatenate(
                [jnp.ones((TBR, d), jnp.float32), x[:, :L - d]], axis=1)
            x = x * xs
            d *= 2
        ls0 = ls0 + x * e0
        ls1 = ls1 + x * e1
    m = jnp.maximum(ls0, ls1)
    mn = jnp.minimum(ls0, ls1)
    lse = m + 0.5 * jnp.log(1.0 + jnp.exp(2.0 * (mn - m)))
    chosen = jnp.where(is1, ls1, ls0)
    out_ref[0, 0, :] = jnp.sum(chosen - lse, axis=1)


def _tc_call(idx_tc, eps):
    f = pl.pallas_call(
        _tc_body,
        grid=(TNB,),
        in_specs=[
            pl.BlockSpec((TBR, L), lambda i: (i, 0)),
            pl.BlockSpec((N, L), lambda i: (0, 0)),
            pl.BlockSpec((N, L), lambda i: (0, 0)),
        ],
        out_specs=pl.BlockSpec((1, 1, TBR), lambda i: (i, 0, 0)),
        out_shape=jax.ShapeDtypeStruct((TNB, 1, TBR), jnp.float32),
    )
    out = f(idx_tc, eps[0], eps[1])
    return out.reshape(B - SPLIT)


def kernel(inputs, eps):
    tc_out = _tc_call(inputs[SPLIT:], eps) if SPLIT < B else None
    sc_out = None
    if SPLIT > 0:
        idx_r = inputs[:SPLIT].reshape(SPLIT * L)
        eps_r = jnp.transpose(eps, (2, 0, 1)).astype(jnp.float32)
        bvec = (jnp.arange(NG * LANES, dtype=jnp.int32) * L)
        sc_out = _sc_call(idx_r, eps_r.reshape(L * 2 * N), bvec)
    if sc_out is None:
        return tc_out
    if tc_out is None:
        return sc_out
    return jnp.concatenate([sc_out, tc_out])


# speedup vs baseline: 1.3407x; 1.3407x over previous
import jax
import jax.numpy as jnp
from jax import lax
from jax.experimental import pallas as pl
from jax.experimental.pallas import tpu as pltpu
from jax.experimental.pallas import tpu_sc as plsc

B = 1024
L = 1024
N = 16
NC, NS, LANES = 2, 16, 16
NW = NC * NS
SPLIT = 512
SC_PROBE = False
RPW = max(SPLIT, NW) // NW
NG = RPW // LANES
TBR = 64
TNB = (B - SPLIT) // TBR


def _tree_sum(xs):
    while len(xs) > 1:
        xs = [xs[i] + xs[i + 1] for i in range(0, len(xs), 2)]
    return xs[0]


def _sc_body(idx_hbm, eps_hbm, bvec_hbm, out_hbm, idx_v, eps_v, out_v,
             eps_sh, bvec_v):
    sid = lax.axis_index("s")
    wid = sid * NC + lax.axis_index("c")
    pltpu.sync_copy(idx_hbm.at[pl.ds(wid * (RPW * L), RPW * L)], idx_v)
    pltpu.sync_copy(bvec_hbm, bvec_v)

    @pl.when(sid == 0)
    def _():
        pltpu.sync_copy(eps_hbm, eps_sh)
    plsc.subcore_barrier()
    pltpu.sync_copy(eps_sh, eps_v)

    def _epilogue(chosen, other, acc):
        m = jnp.maximum(chosen, other)
        mn = jnp.minimum(chosen, other)
        y = jnp.exp(2.0 * (mn - m))
        z = y / (2.0 + y)
        z2 = z * z
        log1p = 2.0 * z * (1.0 + z2 * (1.0 / 3 + z2 * (1.0 / 5 + z2 * (
            1.0 / 7))))
        return acc + (chosen - (m + 0.5 * log1p))

    def _esel_base(t, bvec):
        srow = plsc.load_gather(idx_v, [bvec + t])
        return srow * N + t * (2 * N)

    def _es_row(t):
        return (eps_v[pl.ds(t * (2 * N), N)] +
                eps_v[pl.ds(t * (2 * N) + N, N)])

    for g in range(NG):
        bvec = bvec_v[pl.ds(g * LANES, LANES)]

        def step(t, carry, g=g, bvec=bvec):
            acc = carry[0]
            base = carry[1]
            ES = carry[2]
            P = list(carry[3:])
            tsum, csum = [], []
            for n in range(N):
                esel = plsc.load_gather(eps_v, [base + n])
                tsum.append(P[n] * ES[n])
                P[n] = P[n] * esel
                csum.append(P[n])
            base_n = _esel_base(t + 1, bvec)
            ES_n = _es_row(t + 1)
            chosen = _tree_sum(csum)
            other = _tree_sum(tsum) - chosen
            acc = _epilogue(chosen, other, acc)
            return (acc, base_n, ES_n, *P)

        ones = jnp.ones((LANES,), jnp.float32)
        zeros = jnp.zeros((LANES,), jnp.float32)
        carry = lax.fori_loop(0, L - 1, step,
                              (zeros, _esel_base(0, bvec), _es_row(0))
                              + (ones,) * N,
                              unroll=8)
        acc, base, ES = carry[0], carry[1], carry[2]
        P = list(carry[3:])
        tsum, csum = [], []
        for n in range(N):
            esel = plsc.load_gather(eps_v, [base + n])
            tsum.append(P[n] * ES[n])
            csum.append(P[n] * esel)
        chosen = _tree_sum(csum)
        other = _tree_sum(tsum) - chosen
        acc = _epilogue(chosen, other, acc)
        out_v[pl.ds(g * LANES, LANES)] = acc

    pltpu.sync_copy(out_v, out_hbm.at[pl.ds(wid * RPW, RPW)])


def _sc_call(idx_r, eps_r, bvec):
    f = pl.kernel(
        _sc_body,
        out_type=jax.ShapeDtypeStruct((SPLIT,), jnp.float32),
        mesh=plsc.VectorSubcoreMesh(core_axis_name="c", subcore_axis_name="s"),
        compiler_params=pltpu.CompilerParams(needs_layout_passes=False, skip_device_barrier=True),
        scratch_types=[
            pltpu.VMEM((L * RPW,), jnp.int32),
            pltpu.VMEM((L * 2 * N,), jnp.float32),
            pltpu.VMEM((RPW,), jnp.float32),
            pltpu.VMEM_SHARED((L * 2 * N,), jnp.float32),
            pltpu.VMEM((NG * LANES,), jnp.int32),
        ],
    )
    return f(idx_r, eps_r, bvec)


def _tc_body(idx_ref, e0_ref, e1_ref, out_ref):
    is1 = idx_ref[...] > 0
    ls0 = jnp.zeros((TBR, L), jnp.float32)
    ls1 = jnp.zeros((TBR, L), jnp.float32)
    for n in range(N):
        e0 = e0_ref[n, :].reshape(1, L)
        e1 = e1_ref[n, :].reshape(1, L)
        x = jnp.where(is1, e1, e0)
        x = jnp.concatenate(
            [jnp.ones((TBR, 1), jnp.float32), x[:, :L - 1]], axis=1)
        d = 1
        while d < L:
            xs = jnp.concatenate(
                [jnp.ones((TBR, d), jnp.float32), x[:, :L - d]], axis=1)
            x = x * xs
            d *= 2
        ls0 = ls0 + x * e0
        ls1 = ls1 + x * e1
    m = jnp.maximum(ls0, ls1)
    mn = jnp.minimum(ls0, ls1)
    lse = m + 0.5 * jnp.log(1.0 + jnp.exp(2.0 * (mn - m)))
    chosen = jnp.where(is1, ls1, ls0)
    out_ref[0, 0, :] = jnp.sum(chosen - lse, axis=1)


def _tc_call(idx_tc, eps):
    f = pl.pallas_call(
        _tc_body,
        grid=(TNB,),
        in_specs=[
            pl.BlockSpec((TBR, L), lambda i: (i, 0)),
            pl.BlockSpec((N, L), lambda i: (0, 0)),
            pl.BlockSpec((N, L), lambda i: (0, 0)),
        ],
        out_specs=pl.BlockSpec((1, 1, TBR), lambda i: (i, 0, 0)),
        out_shape=jax.ShapeDtypeStruct((TNB, 1, TBR), jnp.float32),
    )
    out = f(idx_tc, eps[0], eps[1])
    return out.reshape(B - SPLIT)


def kernel(inputs, eps):
    tc_out = _tc_call(inputs[SPLIT:], eps) if SPLIT < B else None
    sc_out = None
    if SPLIT > 0:
        idx_r = inputs[:SPLIT].reshape(SPLIT * L)
        eps_r = jnp.transpose(eps, (2, 0, 1)).astype(jnp.float32)
        bvec = (jnp.arange(NG * LANES, dtype=jnp.int32) * L)
        sc_out = _sc_call(idx_r, eps_r.reshape(L * 2 * N), bvec)
    if sc_out is None:
        return tc_out
    if tc_out is None:
        return sc_out
    return jnp.concatenate([sc_out, tc_out])


# speedup vs baseline: 1.3734x; 1.0243x over previous
import jax
import jax.numpy as jnp
from jax import lax
from jax.experimental import pallas as pl
from jax.experimental.pallas import tpu as pltpu
from jax.experimental.pallas import tpu_sc as plsc

B = 1024
L = 1024
N = 16
NC, NS, LANES = 2, 16, 16
NW = NC * NS
SPLIT = 512
SC_PROBE = False
RPW = max(SPLIT, NW) // NW
NG = RPW // LANES
TBR = 128
TNB = (B - SPLIT) // TBR


def _tree_sum(xs):
    while len(xs) > 1:
        xs = [xs[i] + xs[i + 1] for i in range(0, len(xs), 2)]
    return xs[0]


def _sc_body(idx_hbm, eps_hbm, bvec_hbm, out_hbm, idx_v, eps_v, out_v,
             eps_sh, bvec_v):
    sid = lax.axis_index("s")
    wid = sid * NC + lax.axis_index("c")
    pltpu.sync_copy(idx_hbm.at[pl.ds(wid * (RPW * L), RPW * L)], idx_v)
    pltpu.sync_copy(bvec_hbm, bvec_v)

    @pl.when(sid == 0)
    def _():
        pltpu.sync_copy(eps_hbm, eps_sh)
    plsc.subcore_barrier()
    pltpu.sync_copy(eps_sh, eps_v)

    def _epilogue(chosen, other, acc):
        m = jnp.maximum(chosen, other)
        mn = jnp.minimum(chosen, other)
        y = jnp.exp(2.0 * (mn - m))
        z = y / (2.0 + y)
        z2 = z * z
        log1p = 2.0 * z * (1.0 + z2 * (1.0 / 3 + z2 * (1.0 / 5 + z2 * (
            1.0 / 7))))
        return acc + (chosen - (m + 0.5 * log1p))

    def _esel_base(t, bvec):
        srow = plsc.load_gather(idx_v, [bvec + t])
        return srow * N + t * (2 * N)

    def _es_row(t):
        return (eps_v[pl.ds(t * (2 * N), N)] +
                eps_v[pl.ds(t * (2 * N) + N, N)])

    for g in range(NG):
        bvec = bvec_v[pl.ds(g * LANES, LANES)]

        def step(t, carry, g=g, bvec=bvec):
            acc = carry[0]
            base = carry[1]
            ES = carry[2]
            P = list(carry[3:])
            tsum, csum = [], []
            for n in range(N):
                esel = plsc.load_gather(eps_v, [base + n])
                tsum.append(P[n] * ES[n])
                P[n] = P[n] * esel
                csum.append(P[n])
            base_n = _esel_base(t + 1, bvec)
            ES_n = _es_row(t + 1)
            chosen = _tree_sum(csum)
            other = _tree_sum(tsum) - chosen
            acc = _epilogue(chosen, other, acc)
            return (acc, base_n, ES_n, *P)

        ones = jnp.ones((LANES,), jnp.float32)
        zeros = jnp.zeros((LANES,), jnp.float32)
        carry = lax.fori_loop(0, L - 1, step,
                              (zeros, _esel_base(0, bvec), _es_row(0))
                              + (ones,) * N,
                              unroll=8)
        acc, base, ES = carry[0], carry[1], carry[2]
        P = list(carry[3:])
        tsum, csum = [], []
        for n in range(N):
            esel = plsc.load_gather(eps_v, [base + n])
            tsum.append(P[n] * ES[n])
            csum.append(P[n] * esel)
        chosen = _tree_sum(csum)
        other = _tree_sum(tsum) - chosen
        acc = _epilogue(chosen, other, acc)
        out_v[pl.ds(g * LANES, LANES)] = acc

    pltpu.sync_copy(out_v, out_hbm.at[pl.ds(wid * RPW, RPW)])


def _sc_call(idx_r, eps_r, bvec):
    f = pl.kernel(
        _sc_body,
        out_type=jax.ShapeDtypeStruct((SPLIT,), jnp.float32),
        mesh=plsc.VectorSubcoreMesh(core_axis_name="c", subcore_axis_name="s"),
        compiler_params=pltpu.CompilerParams(needs_layout_passes=False, skip_device_barrier=True),
        scratch_types=[
            pltpu.VMEM((L * RPW,), jnp.int32),
            pltpu.VMEM((L * 2 * N,), jnp.float32),
            pltpu.VMEM((RPW,), jnp.float32),
            pltpu.VMEM_SHARED((L * 2 * N,), jnp.float32),
            pltpu.VMEM((NG * LANES,), jnp.int32),
        ],
    )
    return f(idx_r, eps_r, bvec)


def _tc_body(idx_ref, e0_ref, e1_ref, out_ref):
    is1 = idx_ref[...] > 0
    ls0 = jnp.zeros((TBR, L), jnp.float32)
    ls1 = jnp.zeros((TBR, L), jnp.float32)
    for n in range(N):
        e0 = e0_ref[n, :].reshape(1, L)
        e1 = e1_ref[n, :].reshape(1, L)
        x = jnp.where(is1, e1, e0)
        x = jnp.concatenate(
            [jnp.ones((TBR, 1), jnp.float32), x[:, :L - 1]], axis=1)
        d = 1
        while d < L:
            xs = jnp.concatenate(
                [jnp.ones((TBR, d), jnp.float32), x[:, :L - d]], axis=1)
            x = x * xs
            d *= 2
        ls0 = ls0 + x * e0
        ls1 = ls1 + x * e1
    m = jnp.maximum(ls0, ls1)
    mn = jnp.minimum(ls0, ls1)
    lse = m + 0.5 * jnp.log(1.0 + jnp.exp(2.0 * (mn - m)))
    chosen = jnp.where(is1, ls1, ls0)
    out_ref[0, 0, :] = jnp.sum(chosen - lse, axis=1)


def _tc_call(idx_tc, eps):
    f = pl.pallas_call(
        _tc_body,
        grid=(TNB,),
        in_specs=[
            pl.BlockSpec((TBR, L), lambda i: (i, 0)),
            pl.BlockSpec((N, L), lambda i: (0, 0)),
            pl.BlockSpec((N, L), lambda i: (0, 0)),
        ],
        out_specs=pl.BlockSpec((1, 1, TBR), lambda i: (i, 0, 0)),
        out_shape=jax.ShapeDtypeStruct((TNB, 1, TBR), jnp.float32),
    )
    out = f(idx_tc, eps[0], eps[1])
    return out.reshape(B - SPLIT)


def kernel(inputs, eps):
    tc_out = _tc_call(inputs[SPLIT:], eps) if SPLIT < B else None
    sc_out = None
    if SPLIT > 0:
        idx_r = inputs[:SPLIT].reshape(SPLIT * L)
        eps_r = jnp.transpose(eps, (2, 0, 1)).astype(jnp.float32)
        bvec = (jnp.arange(NG * LANES, dtype=jnp.int32) * L)
        sc_out = _sc_call(idx_r, eps_r.reshape(L * 2 * N), bvec)
    if sc_out is None:
        return tc_out
    if tc_out is None:
        return sc_out
    return jnp.concatenate([sc_out, tc_out])
